# Initial kernel scaffold; baseline (speedup 1.0000x reference)
import functools

import jax
import jax.numpy as jnp
from jax import lax
from jax.experimental import pallas as pl
from jax.experimental.pallas import tpu as pltpu
from jax.experimental.pallas import tpu_sc as plsc

N = 320000
D = 128
H = 128
B = 1024

NW = 32
CROWS = N // NW
T = 400
NT = CROWS // T

MLP_R = 512


def _mlp_body(x_ref, posp_ref, w1x_ref, w1p_ref, b1_ref, w2_ref, b2_ref, o_ref):
    h = jnp.dot(x_ref[...], w1x_ref[...], preferred_element_type=jnp.float32)
    h += jnp.dot(posp_ref[...], w1p_ref[...], preferred_element_type=jnp.float32)
    h = jnp.maximum(h + b1_ref[...], 0.0)
    h = jnp.dot(h, w2_ref[...], preferred_element_type=jnp.float32)
    o_ref[...] = h + b2_ref[...]


def _mlp(x, posp, w1x, w1p, b1, w2, b2):
    grid = (N // MLP_R,)
    return pl.pallas_call(
        _mlp_body,
        grid=grid,
        in_specs=[
            pl.BlockSpec((MLP_R, D), lambda k: (k, 0)),
            pl.BlockSpec((MLP_R, 8), lambda k: (k, 0)),
            pl.BlockSpec((D, H), lambda k: (0, 0)),
            pl.BlockSpec((8, H), lambda k: (0, 0)),
            pl.BlockSpec((1, H), lambda k: (0, 0)),
            pl.BlockSpec((H, H), lambda k: (0, 0)),
            pl.BlockSpec((1, H), lambda k: (0, 0)),
        ],
        out_specs=pl.BlockSpec((MLP_R, H), lambda k: (k, 0)),
        out_shape=jax.ShapeDtypeStruct((N, H), jnp.float32),
    )(x, posp, w1x, w1p, b1, w2, b2)


_NEG = jnp.float32(jnp.finfo(jnp.float32).min)


def _seg_phase1_body(h_hbm, ids_hbm, o1_hbm, side_hbm, data_v, ids_v, stage_v, sem):
    cid = lax.axis_index("c")
    sid = lax.axis_index("s")
    wid = sid * 2 + cid
    base = wid * CROWS

    def flush(prev, fid, m):
        for k in range(8):
            stage_v[pl.ds(k * 16, 16)] = m[k]

        def to_side():
            pltpu.sync_copy(stage_v, side_hbm.at[wid])

        def to_out():
            pltpu.sync_copy(stage_v, o1_hbm.at[prev])

        lax.cond(prev == fid, to_side, to_out)

    def tile_loop(t, carry):
        r0 = base + t * T
        pltpu.sync_copy(h_hbm.at[pl.ds(r0, T)], data_v)
        pltpu.sync_copy(ids_hbm.at[pl.ds(r0, T)], ids_v)

        def row_loop(r, rcarry):
            prev, fid = rcarry[0], rcarry[1]
            m = list(rcarry[2:])
            i = ids_v[r]
            d = [data_v[r, pl.ds(k * 16, 16)] for k in range(8)]
            first = fid < 0
            fid2 = jnp.where(first, i, fid)
            prev2 = jnp.where(first, i, prev)
            changed = i != prev2

            def on_change():
                flush(prev2, fid2, m)
                return tuple(d)

            def on_same():
                return tuple(jnp.maximum(m[k], d[k]) for k in range(8))

            mnew = lax.cond(changed, on_change, on_same)
            return (i, fid2) + tuple(mnew)

        return lax.fori_loop(0, T, row_loop, carry)

    init = (jnp.int32(-1), jnp.int32(-1)) + tuple(
        jnp.full((16,), _NEG, jnp.float32) for _ in range(8)
    )
    final = lax.fori_loop(0, NT, tile_loop, init)
    prev, fid = final[0], final[1]
    m = list(final[2:])
    flush(prev, fid, m)


def _seg_phase1(h, ids):
    mesh = plsc.VectorSubcoreMesh(core_axis_name="c", subcore_axis_name="s")
    f = pl.kernel(
        _seg_phase1_body,
        out_type=[
            jax.ShapeDtypeStruct((B, H), jnp.float32),
            jax.ShapeDtypeStruct((NW, H), jnp.float32),
        ],
        mesh=mesh,
        scratch_types=[
            pltpu.VMEM((T, H), jnp.float32),
            pltpu.VMEM((T,), jnp.int32),
            pltpu.VMEM((H,), jnp.float32),
            pltpu.SemaphoreType.DMA,
        ],
    )
    return f(h, ids)


RPW = B // NW


def _seg_phase2_body(o1_hbm, side_hbm, code_hbm, wlo_hbm, whi_hbm, o2_hbm,
                     o1_v, side_v, code_v, wlo_v, whi_v, out_v, sem):
    cid = lax.axis_index("c")
    sid = lax.axis_index("s")
    wid = sid * 2 + cid
    base = wid * RPW

    pltpu.sync_copy(o1_hbm.at[pl.ds(base, RPW)], o1_v)
    pltpu.sync_copy(side_hbm, side_v)
    pltpu.sync_copy(code_hbm.at[pl.ds(base, RPW)], code_v)
    pltpu.sync_copy(wlo_hbm.at[pl.ds(base, RPW)], wlo_v)
    pltpu.sync_copy(whi_hbm.at[pl.ds(base, RPW)], whi_v)

    def row_loop(j, _):
        c = code_v[j]
        lo = wlo_v[j]
        hi = whi_v[j]
        val = []
        for k in range(8):
            o1k = o1_v[j, pl.ds(k * 16, 16)]
            v = jnp.where(c == 1, o1k,
                          jnp.where(c == 0, jnp.zeros((16,), jnp.float32),
                                    jnp.full((16,), _NEG, jnp.float32)))
            val.append(v)

        def side_loop(w, vcarry):
            return tuple(
                jnp.maximum(vcarry[k], side_v[w, pl.ds(k * 16, 16)])
                for k in range(8)
            )

        val = lax.fori_loop(lo, hi, side_loop, tuple(val))
        for k in range(8):
            out_v[j, pl.ds(k * 16, 16)] = val[k]
        return 0

    lax.fori_loop(0, RPW, row_loop, 0)
    pltpu.sync_copy(out_v, o2_hbm.at[pl.ds(base, RPW)])


def _seg_phase2(o1, side, code, wlo, whi):
    mesh = plsc.VectorSubcoreMesh(core_axis_name="c", subcore_axis_name="s")
    f = pl.kernel(
        _seg_phase2_body,
        out_type=jax.ShapeDtypeStruct((B, H), jnp.float32),
        mesh=mesh,
        scratch_types=[
            pltpu.VMEM((RPW, H), jnp.float32),
            pltpu.VMEM((NW, H), jnp.float32),
            pltpu.VMEM((RPW,), jnp.int32),
            pltpu.VMEM((RPW,), jnp.int32),
            pltpu.VMEM((RPW,), jnp.int32),
            pltpu.VMEM((RPW, H), jnp.float32),
            pltpu.SemaphoreType.DMA,
        ],
    )
    return f(o1, side, code, wlo, whi)


@jax.jit
def _run(x, pos, batch, W1, b1, W2, b2):
    posp = jnp.zeros((N, 8), jnp.float32).at[:, :3].set(pos)
    w1x = W1[:D]
    w1p = jnp.zeros((8, H), jnp.float32).at[:3].set(W1[D:])
    b1r = b1.reshape(1, H)
    b2r = b2.reshape(1, H)

    h = _mlp(x, posp, w1x, w1p, b1r, w2=W2, b2=b2r)

    ids = batch.astype(jnp.int32)
    cw = jnp.arange(NW, dtype=jnp.int32) * CROWS
    fids = ids[cw]
    lids = ids[cw + CROWS - 1]
    s = jnp.arange(B, dtype=jnp.int32)
    directly = jnp.any((fids[None, :] < s[:, None]) & (s[:, None] <= lids[None, :]),
                       axis=1)
    splits = jnp.searchsorted(ids, jnp.arange(B + 1, dtype=jnp.int32))
    empty = splits[1:] == splits[:-1]
    code = jnp.where(empty, 0, jnp.where(directly, 1, 2)).astype(jnp.int32)
    wlo = jnp.searchsorted(fids, s, side="left").astype(jnp.int32)
    whi = jnp.searchsorted(fids, s, side="right").astype(jnp.int32)

    o1, side = _seg_phase1(h, ids)
    out = _seg_phase2(o1, side, code, wlo, whi)

    pos_out = jnp.zeros((B, 3), dtype=pos.dtype)
    batch_out = jnp.arange(B, dtype=batch.dtype)
    return (out, pos_out, batch_out)


def kernel(x, pos, batch, W1, b1, W2, b2):
    return _run(x, pos, batch, W1, b1, W2, b2)


# speedup vs baseline: 1.2278x; 1.2278x over previous
import functools

import jax
import jax.numpy as jnp
from jax import lax
from jax.experimental import pallas as pl
from jax.experimental.pallas import tpu as pltpu
from jax.experimental.pallas import tpu_sc as plsc

N = 320000
D = 128
H = 128
B = 1024

NW = 32
CROWS = N // NW
T = 400
NT = CROWS // T

MLP_R = 2048


def _mlp_body(x_ref, posp_ref, w1x_ref, w1p_ref, b1_ref, w2_ref, b2_ref, o_ref):
    h = jnp.dot(x_ref[...], w1x_ref[...], preferred_element_type=jnp.float32)
    h += jnp.dot(posp_ref[...], w1p_ref[...], preferred_element_type=jnp.float32)
    h = jnp.maximum(h + b1_ref[...], 0.0)
    h = jnp.dot(h, w2_ref[...], preferred_element_type=jnp.float32)
    o_ref[...] = h + b2_ref[...]


def _mlp(x, posp, w1x, w1p, b1, w2, b2):
    grid = (N // MLP_R,)
    return pl.pallas_call(
        _mlp_body,
        grid=grid,
        in_specs=[
            pl.BlockSpec((MLP_R, D), lambda k: (k, 0)),
            pl.BlockSpec((MLP_R, 8), lambda k: (k, 0)),
            pl.BlockSpec((D, H), lambda k: (0, 0)),
            pl.BlockSpec((8, H), lambda k: (0, 0)),
            pl.BlockSpec((1, H), lambda k: (0, 0)),
            pl.BlockSpec((H, H), lambda k: (0, 0)),
            pl.BlockSpec((1, H), lambda k: (0, 0)),
        ],
        out_specs=pl.BlockSpec((MLP_R, H), lambda k: (k, 0)),
        out_shape=jax.ShapeDtypeStruct((N, H), jnp.float32),
    )(x, posp, w1x, w1p, b1, w2, b2)


_NEG = float(jnp.finfo(jnp.float32).min)


def _seg_phase1_body(h_hbm, ids_hbm, o1_hbm, side_hbm, data_v, ids_v, stage_v, sem):
    cid = lax.axis_index("c")
    sid = lax.axis_index("s")
    wid = sid * 2 + cid
    base = wid * CROWS

    def flush(prev, fid, m):
        for k in range(8):
            stage_v[pl.ds(k * 16, 16)] = m[k]

        def to_side():
            pltpu.sync_copy(stage_v, side_hbm.at[wid])

        def to_out():
            pltpu.sync_copy(stage_v, o1_hbm.at[prev])

        lax.cond(prev == fid, to_side, to_out)

    def tile_loop(t, carry):
        r0 = base + t * T
        pltpu.sync_copy(h_hbm.at[pl.ds(r0, T)], data_v)
        pltpu.sync_copy(ids_hbm.at[pl.ds(r0, T)], ids_v)

        def group_loop(q, gcarry):
            prev, fid = gcarry[0], gcarry[1]
            m = list(gcarry[2:])
            ids16 = ids_v[pl.ds(q * 16, 16)]
            for j in range(16):
                r = q * 16 + j
                i = ids16[j]
                d = [data_v[r, pl.ds(k * 16, 16)] for k in range(8)]
                first = fid < 0
                fid = jnp.where(first, i, fid)
                prev = jnp.where(first, i, prev)
                changed = i != prev

                def on_change(prev=prev, fid=fid, m=m):
                    flush(prev, fid, m)

                lax.cond(changed, on_change, lambda: None)
                neg = jnp.full((16,), _NEG, jnp.float32)
                m = [jnp.maximum(jnp.where(changed, neg, m[k]), d[k])
                     for k in range(8)]
                prev = i
            return (prev, fid) + tuple(m)

        return lax.fori_loop(0, T // 16, group_loop, carry)

    init = (jnp.int32(-1), jnp.int32(-1)) + tuple(
        jnp.full((16,), _NEG, jnp.float32) for _ in range(8)
    )
    final = lax.fori_loop(0, NT, tile_loop, init)
    prev, fid = final[0], final[1]
    m = list(final[2:])
    flush(prev, fid, m)


def _seg_phase1(h, ids):
    mesh = plsc.VectorSubcoreMesh(core_axis_name="c", subcore_axis_name="s")
    f = pl.kernel(
        _seg_phase1_body,
        out_type=[
            jax.ShapeDtypeStruct((B, H), jnp.float32),
            jax.ShapeDtypeStruct((NW, H), jnp.float32),
        ],
        mesh=mesh,
        scratch_types=[
            pltpu.VMEM((T, H), jnp.float32),
            pltpu.VMEM((T,), jnp.int32),
            pltpu.VMEM((H,), jnp.float32),
            pltpu.SemaphoreType.DMA,
        ],
    )
    return f(h, ids)


RPW = B // NW


def _seg_phase2_body(o1_hbm, side_hbm, code_hbm, wlo_hbm, whi_hbm, o2_hbm,
                     o1_v, side_v, code_v, wlo_v, whi_v, out_v, sem):
    cid = lax.axis_index("c")
    sid = lax.axis_index("s")
    wid = sid * 2 + cid
    base = wid * RPW

    pltpu.sync_copy(o1_hbm.at[pl.ds(base, RPW)], o1_v)
    pltpu.sync_copy(side_hbm, side_v)
    pltpu.sync_copy(code_hbm.at[pl.ds(base, RPW)], code_v)
    pltpu.sync_copy(wlo_hbm.at[pl.ds(base, RPW)], wlo_v)
    pltpu.sync_copy(whi_hbm.at[pl.ds(base, RPW)], whi_v)

    for q in range(RPW // 16):
        code16 = code_v[pl.ds(q * 16, 16)]
        wlo16 = wlo_v[pl.ds(q * 16, 16)]
        whi16 = whi_v[pl.ds(q * 16, 16)]
        for j in range(16):
            row = q * 16 + j
            c = code16[j]
            lo = wlo16[j]
            hi = whi16[j]
            val = []
            for k in range(8):
                o1k = o1_v[row, pl.ds(k * 16, 16)]
                v = jnp.where(c == 1, o1k,
                              jnp.where(c == 0, jnp.zeros((16,), jnp.float32),
                                        jnp.full((16,), _NEG, jnp.float32)))
                val.append(v)

            def side_loop(w, vcarry):
                return tuple(
                    jnp.maximum(vcarry[k], side_v[w, pl.ds(k * 16, 16)])
                    for k in range(8)
                )

            val = lax.fori_loop(lo, hi, side_loop, tuple(val))
            for k in range(8):
                out_v[row, pl.ds(k * 16, 16)] = val[k]

    pltpu.sync_copy(out_v, o2_hbm.at[pl.ds(base, RPW)])


def _seg_phase2(o1, side, code, wlo, whi):
    mesh = plsc.VectorSubcoreMesh(core_axis_name="c", subcore_axis_name="s")
    f = pl.kernel(
        _seg_phase2_body,
        out_type=jax.ShapeDtypeStruct((B, H), jnp.float32),
        mesh=mesh,
        scratch_types=[
            pltpu.VMEM((RPW, H), jnp.float32),
            pltpu.VMEM((NW, H), jnp.float32),
            pltpu.VMEM((RPW,), jnp.int32),
            pltpu.VMEM((RPW,), jnp.int32),
            pltpu.VMEM((RPW,), jnp.int32),
            pltpu.VMEM((RPW, H), jnp.float32),
            pltpu.SemaphoreType.DMA,
        ],
    )
    return f(o1, side, code, wlo, whi)


@jax.jit
def _run(x, pos, batch, W1, b1, W2, b2):
    posp = jnp.zeros((N, 8), jnp.float32).at[:, :3].set(pos)
    w1x = W1[:D]
    w1p = jnp.zeros((8, H), jnp.float32).at[:3].set(W1[D:])
    b1r = b1.reshape(1, H)
    b2r = b2.reshape(1, H)

    h = _mlp(x, posp, w1x, w1p, b1r, w2=W2, b2=b2r)

    ids = batch.astype(jnp.int32)
    cw = jnp.arange(NW, dtype=jnp.int32) * CROWS
    fids = ids[cw]
    lids = ids[cw + CROWS - 1]
    s = jnp.arange(B, dtype=jnp.int32)
    directly = jnp.any((fids[None, :] < s[:, None]) & (s[:, None] <= lids[None, :]),
                       axis=1)
    wlo = jnp.searchsorted(fids, s, side="left").astype(jnp.int32)
    whi = jnp.searchsorted(fids, s, side="right").astype(jnp.int32)
    nonempty = directly | (wlo < whi)
    code = jnp.where(nonempty, jnp.where(directly, 1, 2), 0).astype(jnp.int32)

    o1, side = _seg_phase1(h, ids)
    out = _seg_phase2(o1, side, code, wlo, whi)

    pos_out = jnp.zeros((B, 3), dtype=pos.dtype)
    batch_out = jnp.arange(B, dtype=batch.dtype)
    return (out, pos_out, batch_out)


def kernel(x, pos, batch, W1, b1, W2, b2):
    return _run(x, pos, batch, W1, b1, W2, b2)
